# baseline (speedup 1.0000x reference)
import functools

import jax
import jax.numpy as jnp
from jax.experimental import pallas as pl
from jax.experimental.pallas import tpu as pltpu

_ALPHA = 0.2
_BR = 256


def _elu(z):
    return jnp.where(z > 0, z, jnp.exp(jnp.minimum(z, 0.0)) - 1.0)


def _gat_kernel(nheads, nhid, nclass, br, nblk,
                x_ref, Wall_ref, Asrc_ref, Adst_ref,
                adj_ref, Wout_ref, a2s_ref, a2d_ref, PvT_ref,
                out_ref,
                haug_s, r_s, vT_s, qT_s, adj16_s,
                h2aug_s, r2_s, v2T_s, q2T_s):
    bf16 = jnp.bfloat16
    i = pl.program_id(0)
    blk = jax.lax.rem(i, nblk)
    row0 = blk * br

    @pl.when(i == 0)
    def _():
        h = jnp.dot(x_ref[...], Wall_ref[...],
                    preferred_element_type=jnp.float32)
        ones = jnp.ones((h.shape[0], 1), jnp.float32)
        for hd in range(nheads):
            haug_s[hd] = jnp.concatenate(
                [h[:, hd * nhid:(hd + 1) * nhid], ones], axis=1).astype(bf16)
        ls = jnp.dot(h, Asrc_ref[...], preferred_element_type=jnp.float32)
        r_s[...] = jnp.exp((1.0 - _ALPHA) * ls).astype(bf16)
        ld = jnp.dot(h, Adst_ref[...], preferred_element_type=jnp.float32)
        ldT = jnp.transpose(ld, (1, 0))
        vT_s[...] = jnp.exp(-ldT).astype(bf16)
        qT_s[...] = jnp.exp(-_ALPHA * ldT).astype(bf16)

    @pl.when(i < nblk)
    def _():
        adj = adj_ref[...].astype(bf16)
        adj16_s[pl.ds(row0, br), :] = adj
        outs = []
        for hd in range(nheads):
            rc = r_s[pl.ds(row0, br), hd:hd + 1]
            vr = vT_s[hd:hd + 1, :]
            qr = qT_s[hd:hd + 1, :]
            e = jnp.minimum(vr, rc * qr) * adj
            hp = jnp.dot(e, haug_s[hd], preferred_element_type=jnp.float32)
            outs.append(_elu(hp[:, :nhid] / hp[:, nhid:nhid + 1]))
        x1 = jnp.concatenate(outs, axis=1)
        h2 = jnp.dot(x1, Wout_ref[...], preferred_element_type=jnp.float32)
        ones = jnp.ones((h2.shape[0], 1), jnp.float32)
        h2aug_s[pl.ds(row0, br), :] = jnp.concatenate(
            [h2, ones], axis=1).astype(bf16)
        ls2 = jnp.dot(h2, a2s_ref[...], preferred_element_type=jnp.float32)
        ld2 = jnp.dot(h2, a2d_ref[...], preferred_element_type=jnp.float32)
        r2_s[pl.ds(row0, br), :] = jnp.exp((1.0 - _ALPHA) * ls2).astype(bf16)
        ld2T = jnp.transpose(ld2, (1, 0))
        v2T_s[0:1, pl.ds(row0, br)] = jnp.exp(-ld2T).astype(bf16)
        q2T_s[0:1, pl.ds(row0, br)] = jnp.exp(-_ALPHA * ld2T).astype(bf16)

    @pl.when(i >= nblk)
    def _():
        adj = adj16_s[pl.ds(row0, br), :]
        e = jnp.minimum(v2T_s[...],
                        r2_s[pl.ds(row0, br), :] * q2T_s[...]) * adj
        hp = jnp.dot(e, h2aug_s[...], preferred_element_type=jnp.float32)
        x2 = _elu(hp[:, :nclass] / hp[:, nclass:nclass + 1])
        contrib = jnp.dot(PvT_ref[:, pl.ds(row0, br)], x2,
                          preferred_element_type=jnp.float32)

        @pl.when(i == nblk)
        def _():
            out_ref[...] = contrib

        @pl.when(i > nblk)
        def _():
            out_ref[...] += contrib

        @pl.when(i == 2 * nblk - 1)
        def _():
            z = out_ref[...]
            m = jnp.max(z, axis=1, keepdims=True)
            zs = z - m
            out_ref[...] = zs - jnp.log(
                jnp.sum(jnp.exp(zs), axis=1, keepdims=True))


def kernel(x, adj, PvT, W_heads, a_heads, W_out, a_out):
    f32 = jnp.float32
    bf16 = jnp.bfloat16
    n, nfeat = x.shape
    nheads, _, nhid = W_heads.shape
    nclass = W_out.shape[1]
    nv = PvT.shape[0]
    fcat = nheads * nhid
    br = _BR if n % _BR == 0 else n
    nblk = n // br

    Wall = jnp.transpose(W_heads, (1, 0, 2)).reshape(nfeat, fcat)
    a_src = a_heads[:, 0, :nhid]
    a_dst = a_heads[:, 0, nhid:]
    eye = jnp.eye(nheads, dtype=f32)
    Asrc = (eye[:, None, :] * a_src[:, :, None]).reshape(fcat, nheads)
    Adst = (eye[:, None, :] * a_dst[:, :, None]).reshape(fcat, nheads)
    a2s = a_out[0, :nclass].reshape(nclass, 1)
    a2d = a_out[0, nclass:].reshape(nclass, 1)

    out = pl.pallas_call(
        functools.partial(_gat_kernel, nheads, nhid, nclass, br, nblk),
        grid=(2 * nblk,),
        in_specs=[
            pl.BlockSpec((n, nfeat), lambda i: (0, 0)),
            pl.BlockSpec((nfeat, fcat), lambda i: (0, 0)),
            pl.BlockSpec((fcat, nheads), lambda i: (0, 0)),
            pl.BlockSpec((fcat, nheads), lambda i: (0, 0)),
            pl.BlockSpec((br, n), lambda i: (jnp.minimum(i, nblk - 1), 0)),
            pl.BlockSpec((fcat, nclass), lambda i: (0, 0)),
            pl.BlockSpec((nclass, 1), lambda i: (0, 0)),
            pl.BlockSpec((nclass, 1), lambda i: (0, 0)),
            pl.BlockSpec((nv, n), lambda i: (0, 0)),
        ],
        out_specs=pl.BlockSpec((nv, nclass), lambda i: (0, 0)),
        out_shape=jax.ShapeDtypeStruct((nv, nclass), f32),
        scratch_shapes=[
            pltpu.VMEM((nheads, n, nhid + 1), bf16),
            pltpu.VMEM((n, nheads), bf16),
            pltpu.VMEM((nheads, n), bf16),
            pltpu.VMEM((nheads, n), bf16),
            pltpu.VMEM((n, n), bf16),
            pltpu.VMEM((n, nclass + 1), bf16),
            pltpu.VMEM((n, 1), bf16),
            pltpu.VMEM((1, n), bf16),
            pltpu.VMEM((1, n), bf16),
        ],
    )(x, Wall, Asrc, Adst, adj, W_out, a2s, a2d, PvT)
    return out


# speedup vs baseline: 1.1341x; 1.1341x over previous
import functools

import jax
import jax.numpy as jnp
from jax.experimental import pallas as pl
from jax.experimental.pallas import tpu as pltpu

_ALPHA = 0.2
_BR = 1024


def _elu(z):
    return jnp.where(z > 0, z, jnp.exp(jnp.minimum(z, 0.0)) - 1.0)


def _gat_kernel(nheads, nhid, nclass, br, nblk,
                x_ref, Wall_ref, Asrc_ref, Adst_ref,
                adj_ref, Wout_ref, a2s_ref, a2d_ref, PvT_ref,
                out_ref,
                haug_s, r_s, vT_s, qT_s, adj16_s,
                h2aug_s, r2_s, v2T_s, q2T_s):
    bf16 = jnp.bfloat16
    i = pl.program_id(0)
    blk = jax.lax.rem(i, nblk)
    row0 = blk * br

    @pl.when(i == 0)
    def _():
        h = jnp.dot(x_ref[...], Wall_ref[...],
                    preferred_element_type=jnp.float32)
        ones = jnp.ones((h.shape[0], 1), jnp.float32)
        for hd in range(nheads):
            haug_s[hd] = jnp.concatenate(
                [h[:, hd * nhid:(hd + 1) * nhid], ones], axis=1).astype(bf16)
        ls = jnp.dot(h, Asrc_ref[...], preferred_element_type=jnp.float32)
        r_s[...] = jnp.exp((1.0 - _ALPHA) * ls).astype(bf16)
        ld = jnp.dot(h, Adst_ref[...], preferred_element_type=jnp.float32)
        ldT = jnp.transpose(ld, (1, 0))
        vT_s[...] = jnp.exp(-ldT).astype(bf16)
        qT_s[...] = jnp.exp(-_ALPHA * ldT).astype(bf16)

    @pl.when(i < nblk)
    def _():
        adj = adj_ref[...].astype(bf16)
        adj16_s[pl.ds(row0, br), :] = adj
        outs = []
        for hd in range(nheads):
            rc = r_s[pl.ds(row0, br), hd:hd + 1]
            vr = vT_s[hd:hd + 1, :]
            qr = qT_s[hd:hd + 1, :]
            e = jnp.minimum(vr, rc * qr) * adj
            hp = jnp.dot(e, haug_s[hd], preferred_element_type=jnp.float32)
            outs.append(_elu(hp[:, :nhid] / hp[:, nhid:nhid + 1]))
        x1 = jnp.concatenate(outs, axis=1)
        h2 = jnp.dot(x1, Wout_ref[...], preferred_element_type=jnp.float32)
        ones = jnp.ones((h2.shape[0], 1), jnp.float32)
        h2aug_s[pl.ds(row0, br), :] = jnp.concatenate(
            [h2, ones], axis=1).astype(bf16)
        ls2 = jnp.dot(h2, a2s_ref[...], preferred_element_type=jnp.float32)
        ld2 = jnp.dot(h2, a2d_ref[...], preferred_element_type=jnp.float32)
        r2_s[pl.ds(row0, br), :] = jnp.exp((1.0 - _ALPHA) * ls2).astype(bf16)
        ld2T = jnp.transpose(ld2, (1, 0))
        v2T_s[0:1, pl.ds(row0, br)] = jnp.exp(-ld2T).astype(bf16)
        q2T_s[0:1, pl.ds(row0, br)] = jnp.exp(-_ALPHA * ld2T).astype(bf16)

    @pl.when(i >= nblk)
    def _():
        adj = adj16_s[pl.ds(row0, br), :]
        e = jnp.minimum(v2T_s[...],
                        r2_s[pl.ds(row0, br), :] * q2T_s[...]) * adj
        hp = jnp.dot(e, h2aug_s[...], preferred_element_type=jnp.float32)
        x2 = _elu(hp[:, :nclass] / hp[:, nclass:nclass + 1])
        contrib = jnp.dot(PvT_ref[:, pl.ds(row0, br)], x2,
                          preferred_element_type=jnp.float32)

        @pl.when(i == nblk)
        def _():
            out_ref[...] = contrib

        @pl.when(i > nblk)
        def _():
            out_ref[...] += contrib

        @pl.when(i == 2 * nblk - 1)
        def _():
            z = out_ref[...]
            m = jnp.max(z, axis=1, keepdims=True)
            zs = z - m
            out_ref[...] = zs - jnp.log(
                jnp.sum(jnp.exp(zs), axis=1, keepdims=True))


def kernel(x, adj, PvT, W_heads, a_heads, W_out, a_out):
    f32 = jnp.float32
    bf16 = jnp.bfloat16
    n, nfeat = x.shape
    nheads, _, nhid = W_heads.shape
    nclass = W_out.shape[1]
    nv = PvT.shape[0]
    fcat = nheads * nhid
    br = _BR if n % _BR == 0 else n
    nblk = n // br

    Wall = jnp.transpose(W_heads, (1, 0, 2)).reshape(nfeat, fcat)
    a_src = a_heads[:, 0, :nhid]
    a_dst = a_heads[:, 0, nhid:]
    eye = jnp.eye(nheads, dtype=f32)
    Asrc = (eye[:, None, :] * a_src[:, :, None]).reshape(fcat, nheads)
    Adst = (eye[:, None, :] * a_dst[:, :, None]).reshape(fcat, nheads)
    a2s = a_out[0, :nclass].reshape(nclass, 1)
    a2d = a_out[0, nclass:].reshape(nclass, 1)

    out = pl.pallas_call(
        functools.partial(_gat_kernel, nheads, nhid, nclass, br, nblk),
        grid=(2 * nblk,),
        in_specs=[
            pl.BlockSpec((n, nfeat), lambda i: (0, 0)),
            pl.BlockSpec((nfeat, fcat), lambda i: (0, 0)),
            pl.BlockSpec((fcat, nheads), lambda i: (0, 0)),
            pl.BlockSpec((fcat, nheads), lambda i: (0, 0)),
            pl.BlockSpec((br, n), lambda i: (jnp.minimum(i, nblk - 1), 0)),
            pl.BlockSpec((fcat, nclass), lambda i: (0, 0)),
            pl.BlockSpec((nclass, 1), lambda i: (0, 0)),
            pl.BlockSpec((nclass, 1), lambda i: (0, 0)),
            pl.BlockSpec((nv, n), lambda i: (0, 0)),
        ],
        out_specs=pl.BlockSpec((nv, nclass), lambda i: (0, 0)),
        out_shape=jax.ShapeDtypeStruct((nv, nclass), f32),
        scratch_shapes=[
            pltpu.VMEM((nheads, n, nhid + 1), bf16),
            pltpu.VMEM((n, nheads), bf16),
            pltpu.VMEM((nheads, n), bf16),
            pltpu.VMEM((nheads, n), bf16),
            pltpu.VMEM((n, n), bf16),
            pltpu.VMEM((n, nclass + 1), bf16),
            pltpu.VMEM((n, 1), bf16),
            pltpu.VMEM((1, n), bf16),
            pltpu.VMEM((1, n), bf16),
        ],
    )(x, Wall, Asrc, Adst, adj, W_out, a2s, a2d, PvT)
    return out


# speedup vs baseline: 1.1659x; 1.0280x over previous
import functools

import jax
import jax.numpy as jnp
from jax.experimental import pallas as pl
from jax.experimental.pallas import tpu as pltpu

_ALPHA = 0.2
_BR = 1024


def _elu(z):
    return jnp.where(z > 0, z, jnp.exp(jnp.minimum(z, 0.0)) - 1.0)


def _gat_kernel(nheads, nhid, nclass, br, nblk,
                x_ref, Wh_ref, ah_ref,
                adj_ref, Wout_ref, aout_ref, PvT_ref,
                out_ref,
                haug_s, r_s, vT_s, qT_s, adj16_s,
                h2aug_s, r2_s, v2T_s, q2T_s,
                Wall_s, Asrc_s, Adst_s):
    bf16 = jnp.bfloat16
    i = pl.program_id(0)
    blk = jax.lax.rem(i, nblk)
    row0 = blk * br
    fcat = nheads * nhid

    @pl.when(i == 0)
    def _():
        Asrc_s[...] = jnp.zeros((fcat, nheads), jnp.float32)
        Adst_s[...] = jnp.zeros((fcat, nheads), jnp.float32)
        for hd in range(nheads):
            Wall_s[:, hd * nhid:(hd + 1) * nhid] = Wh_ref[hd]
            Asrc_s[hd * nhid:(hd + 1) * nhid, hd:hd + 1] = jnp.transpose(
                ah_ref[hd, :, :nhid], (1, 0))
            Adst_s[hd * nhid:(hd + 1) * nhid, hd:hd + 1] = jnp.transpose(
                ah_ref[hd, :, nhid:], (1, 0))
        h = jnp.dot(x_ref[...], Wall_s[...],
                    preferred_element_type=jnp.float32)
        ones = jnp.ones((h.shape[0], 1), jnp.float32)
        for hd in range(nheads):
            haug_s[hd] = jnp.concatenate(
                [h[:, hd * nhid:(hd + 1) * nhid], ones], axis=1).astype(bf16)
        ls = jnp.dot(h, Asrc_s[...], preferred_element_type=jnp.float32)
        r_s[...] = jnp.exp((1.0 - _ALPHA) * ls).astype(bf16)
        ld = jnp.dot(h, Adst_s[...], preferred_element_type=jnp.float32)
        ldT = jnp.transpose(ld, (1, 0))
        vT_s[...] = jnp.exp(-ldT).astype(bf16)
        qT_s[...] = jnp.exp(-_ALPHA * ldT).astype(bf16)

    @pl.when(i < nblk)
    def _():
        adj = adj_ref[...].astype(bf16)
        adj16_s[pl.ds(row0, br), :] = adj
        outs = []
        for hd in range(nheads):
            rc = r_s[pl.ds(row0, br), hd:hd + 1]
            vr = vT_s[hd:hd + 1, :]
            qr = qT_s[hd:hd + 1, :]
            e = jnp.minimum(vr, rc * qr) * adj
            hp = jnp.dot(e, haug_s[hd], preferred_element_type=jnp.float32)
            outs.append(_elu(hp[:, :nhid] / hp[:, nhid:nhid + 1]))
        x1 = jnp.concatenate(outs, axis=1)
        h2 = jnp.dot(x1, Wout_ref[...], preferred_element_type=jnp.float32)
        ones = jnp.ones((h2.shape[0], 1), jnp.float32)
        h2aug_s[pl.ds(row0, br), :] = jnp.concatenate(
            [h2, ones], axis=1).astype(bf16)
        a2s = jnp.transpose(aout_ref[:, :nclass], (1, 0))
        a2d = jnp.transpose(aout_ref[:, nclass:], (1, 0))
        ls2 = jnp.dot(h2, a2s, preferred_element_type=jnp.float32)
        ld2 = jnp.dot(h2, a2d, preferred_element_type=jnp.float32)
        r2_s[pl.ds(row0, br), :] = jnp.exp((1.0 - _ALPHA) * ls2).astype(bf16)
        ld2T = jnp.transpose(ld2, (1, 0))
        v2T_s[0:1, pl.ds(row0, br)] = jnp.exp(-ld2T).astype(bf16)
        q2T_s[0:1, pl.ds(row0, br)] = jnp.exp(-_ALPHA * ld2T).astype(bf16)

    @pl.when(i >= nblk)
    def _():
        adj = adj16_s[pl.ds(row0, br), :]
        e = jnp.minimum(v2T_s[...],
                        r2_s[pl.ds(row0, br), :] * q2T_s[...]) * adj
        hp = jnp.dot(e, h2aug_s[...], preferred_element_type=jnp.float32)
        x2 = _elu(hp[:, :nclass] / hp[:, nclass:nclass + 1])
        contrib = jnp.dot(PvT_ref[:, pl.ds(row0, br)], x2,
                          preferred_element_type=jnp.float32)

        @pl.when(i == nblk)
        def _():
            out_ref[...] = contrib

        @pl.when(i > nblk)
        def _():
            out_ref[...] += contrib

        @pl.when(i == 2 * nblk - 1)
        def _():
            z = out_ref[...]
            m = jnp.max(z, axis=1, keepdims=True)
            zs = z - m
            out_ref[...] = zs - jnp.log(
                jnp.sum(jnp.exp(zs), axis=1, keepdims=True))


def kernel(x, adj, PvT, W_heads, a_heads, W_out, a_out):
    f32 = jnp.float32
    bf16 = jnp.bfloat16
    n, nfeat = x.shape
    nheads, _, nhid = W_heads.shape
    nclass = W_out.shape[1]
    nv = PvT.shape[0]
    fcat = nheads * nhid
    br = _BR if n % _BR == 0 else n
    nblk = n // br

    out = pl.pallas_call(
        functools.partial(_gat_kernel, nheads, nhid, nclass, br, nblk),
        grid=(2 * nblk,),
        in_specs=[
            pl.BlockSpec((n, nfeat), lambda i: (0, 0)),
            pl.BlockSpec((nheads, nfeat, nhid), lambda i: (0, 0, 0)),
            pl.BlockSpec((nheads, 1, 2 * nhid), lambda i: (0, 0, 0)),
            pl.BlockSpec((br, n), lambda i: (jnp.minimum(i, nblk - 1), 0)),
            pl.BlockSpec((fcat, nclass), lambda i: (0, 0)),
            pl.BlockSpec((1, 2 * nclass), lambda i: (0, 0)),
            pl.BlockSpec((nv, n), lambda i: (0, 0)),
        ],
        out_specs=pl.BlockSpec((nv, nclass), lambda i: (0, 0)),
        out_shape=jax.ShapeDtypeStruct((nv, nclass), f32),
        scratch_shapes=[
            pltpu.VMEM((nheads, n, nhid + 1), bf16),
            pltpu.VMEM((n, nheads), bf16),
            pltpu.VMEM((nheads, n), bf16),
            pltpu.VMEM((nheads, n), bf16),
            pltpu.VMEM((n, n), bf16),
            pltpu.VMEM((n, nclass + 1), bf16),
            pltpu.VMEM((n, 1), bf16),
            pltpu.VMEM((1, n), bf16),
            pltpu.VMEM((1, n), bf16),
            pltpu.VMEM((nfeat, fcat), f32),
            pltpu.VMEM((fcat, nheads), f32),
            pltpu.VMEM((fcat, nheads), f32),
        ],
    )(x, W_heads, a_heads, adj, W_out, a_out, PvT)
    return out
